# baseline (speedup 1.0000x reference)
import jax
import jax.numpy as jnp
from jax import lax
from jax.experimental import pallas as pl
from jax.experimental.pallas import tpu as pltpu


_VMEM_LIMIT = 64 * 1024 * 1024


def _round_up(x, m):
    return (x + m - 1) // m * m


def _support_body(x_ref, w_ref, s_ref):
    s_ref[...] = jnp.dot(
        x_ref[...].astype(jnp.bfloat16), w_ref[...],
        preferred_element_type=jnp.float32,
    ).astype(jnp.bfloat16)


def _support(x, w_bf16, *, tile):
    n, f = x.shape
    h = w_bf16.shape[1]
    return pl.pallas_call(
        _support_body,
        out_shape=jax.ShapeDtypeStruct((n, h), jnp.bfloat16),
        grid=(n // tile,),
        in_specs=[
            pl.BlockSpec((tile, f), lambda i: (i, 0)),
            pl.BlockSpec((f, h), lambda i: (0, 0)),
        ],
        out_specs=pl.BlockSpec((tile, h), lambda i: (i, 0)),
        compiler_params=pltpu.CompilerParams(
            dimension_semantics=("parallel",),
            vmem_limit_bytes=_VMEM_LIMIT,
        ),
    )(x, w_bf16)


def _z_body(adj_ref, s_ref, z_ref):
    z_ref[...] = jnp.maximum(
        jnp.dot(
            adj_ref[...].astype(jnp.bfloat16), s_ref[...],
            preferred_element_type=jnp.float32,
        ),
        0.0,
    ).astype(jnp.bfloat16)


def _z_pallas(adj, s, *, tile_i):
    n = adj.shape[0]
    h = s.shape[1]
    return pl.pallas_call(
        _z_body,
        out_shape=jax.ShapeDtypeStruct((n, h), jnp.bfloat16),
        grid=(n // tile_i,),
        in_specs=[
            pl.BlockSpec((tile_i, n), lambda i: (i, 0)),
            pl.BlockSpec((n, h), lambda i: (0, 0)),
        ],
        out_specs=pl.BlockSpec((tile_i, h), lambda i: (i, 0)),
        compiler_params=pltpu.CompilerParams(
            dimension_semantics=("parallel",),
            vmem_limit_bytes=_VMEM_LIMIT,
        ),
    )(adj, s)


def _dec_body(zr_ref, zc_ref, o_ref):
    logits = lax.dot_general(
        zr_ref[...], zc_ref[...],
        dimension_numbers=(((1,), (1,)), ((), ())),
        preferred_element_type=jnp.float32,
    )
    o_ref[...] = jax.nn.sigmoid(logits)


def _decoder(z, *, tile_i, tile_j):
    n, h = z.shape
    return pl.pallas_call(
        _dec_body,
        out_shape=jax.ShapeDtypeStruct((n, n), jnp.float32),
        grid=(n // tile_i, n // tile_j),
        in_specs=[
            pl.BlockSpec((tile_i, h), lambda i, j: (i, 0)),
            pl.BlockSpec((tile_j, h), lambda i, j: (j, 0)),
        ],
        out_specs=pl.BlockSpec((tile_i, tile_j), lambda i, j: (i, j)),
        compiler_params=pltpu.CompilerParams(
            dimension_semantics=("parallel", "parallel"),
            vmem_limit_bytes=_VMEM_LIMIT,
        ),
    )(z, z)


def kernel(atac_feature, scrna_feature, adj_norm, edge_attr, gc1_weight):
    del atac_feature, edge_attr

    n = adj_norm.shape[0]
    x = scrna_feature.astype(jnp.float32)
    adj = adj_norm.astype(jnp.float32)
    w_bf16 = gc1_weight.astype(jnp.bfloat16)

    pad = _round_up(n, 1024) - n
    if pad:
        adj = jnp.pad(adj, ((0, pad), (0, pad)))
        x = jnp.pad(x, ((0, pad), (0, 0)))
    n_p = n + pad

    s = _support(x, w_bf16, tile=n_p // 2)
    z = _z_pallas(adj, s, tile_i=n_p // 16)
    a_pred = _decoder(z, tile_i=2048, tile_j=1024)
    return a_pred[:n, :n]


# speedup vs baseline: 1.0397x; 1.0397x over previous
import jax
import jax.numpy as jnp
from jax import lax
from jax.experimental import pallas as pl
from jax.experimental.pallas import tpu as pltpu


_VMEM_LIMIT = 64 * 1024 * 1024


def _round_up(x, m):
    return (x + m - 1) // m * m


def _support_body(x_ref, w_ref, s_ref):
    s_ref[...] = jnp.dot(
        x_ref[...].astype(jnp.bfloat16), w_ref[...],
        preferred_element_type=jnp.float32,
    ).astype(jnp.bfloat16)


def _support(x, w_bf16, *, tile):
    n, f = x.shape
    h = w_bf16.shape[1]
    return pl.pallas_call(
        _support_body,
        out_shape=jax.ShapeDtypeStruct((n, h), jnp.bfloat16),
        grid=(n // tile,),
        in_specs=[
            pl.BlockSpec((tile, f), lambda i: (i, 0)),
            pl.BlockSpec((f, h), lambda i: (0, 0)),
        ],
        out_specs=pl.BlockSpec((tile, h), lambda i: (i, 0)),
        compiler_params=pltpu.CompilerParams(
            dimension_semantics=("parallel",),
            vmem_limit_bytes=_VMEM_LIMIT,
        ),
    )(x, w_bf16)


def _z_body(adj_ref, s_ref, z_ref):
    z_ref[...] = jnp.maximum(
        jnp.dot(
            adj_ref[...].astype(jnp.bfloat16), s_ref[...],
            preferred_element_type=jnp.float32,
        ),
        0.0,
    ).astype(jnp.bfloat16)


def _z_pallas(adj, s, *, tile_i):
    n = adj.shape[0]
    h = s.shape[1]
    return pl.pallas_call(
        _z_body,
        out_shape=jax.ShapeDtypeStruct((n, h), jnp.bfloat16),
        grid=(n // tile_i,),
        in_specs=[
            pl.BlockSpec((tile_i, n), lambda i: (i, 0)),
            pl.BlockSpec((n, h), lambda i: (0, 0)),
        ],
        out_specs=pl.BlockSpec((tile_i, h), lambda i: (i, 0)),
        compiler_params=pltpu.CompilerParams(
            dimension_semantics=("parallel",),
            vmem_limit_bytes=_VMEM_LIMIT,
        ),
    )(adj, s)


def _dec_body(zr_ref, zc_ref, o_ref):
    logits = lax.dot_general(
        zr_ref[...], zc_ref[...],
        dimension_numbers=(((1,), (1,)), ((), ())),
        preferred_element_type=jnp.float32,
    )
    o_ref[...] = jax.nn.sigmoid(logits)


def _decoder(z, *, tile_i, tile_j):
    n, h = z.shape
    return pl.pallas_call(
        _dec_body,
        out_shape=jax.ShapeDtypeStruct((n, n), jnp.float32),
        grid=(n // tile_i, n // tile_j),
        in_specs=[
            pl.BlockSpec((tile_i, h), lambda i, j: (i, 0)),
            pl.BlockSpec((tile_j, h), lambda i, j: (j, 0)),
        ],
        out_specs=pl.BlockSpec((tile_i, tile_j), lambda i, j: (i, j)),
        compiler_params=pltpu.CompilerParams(
            dimension_semantics=("parallel", "parallel"),
            vmem_limit_bytes=_VMEM_LIMIT,
        ),
    )(z, z)


def kernel(atac_feature, scrna_feature, adj_norm, edge_attr, gc1_weight):
    del atac_feature, edge_attr

    n = adj_norm.shape[0]
    x = scrna_feature.astype(jnp.float32)
    adj = adj_norm.astype(jnp.float32)
    w_bf16 = gc1_weight.astype(jnp.bfloat16)

    pad = _round_up(n, 1024) - n
    if pad:
        adj = jnp.pad(adj, ((0, pad), (0, pad)))
        x = jnp.pad(x, ((0, pad), (0, 0)))
    n_p = n + pad

    s = _support(x, w_bf16, tile=n_p // 2)
    z = _z_pallas(adj, s, tile_i=n_p // 4)
    a_pred = _decoder(z, tile_i=2048, tile_j=1024)
    return a_pred[:n, :n]


# speedup vs baseline: 1.0815x; 1.0401x over previous
import jax
import jax.numpy as jnp
from jax import lax
from jax.experimental import pallas as pl
from jax.experimental.pallas import tpu as pltpu


_VMEM_LIMIT = 64 * 1024 * 1024


def _round_up(x, m):
    return (x + m - 1) // m * m


def _support_body(x_ref, w_ref, s_ref):
    s_ref[...] = jnp.dot(
        x_ref[...].astype(jnp.bfloat16), w_ref[...],
        preferred_element_type=jnp.float32,
    ).astype(jnp.bfloat16)


def _support(x, w_bf16, *, tile):
    n, f = x.shape
    h = w_bf16.shape[1]
    return pl.pallas_call(
        _support_body,
        out_shape=jax.ShapeDtypeStruct((n, h), jnp.bfloat16),
        grid=(n // tile,),
        in_specs=[
            pl.BlockSpec((tile, f), lambda i: (i, 0)),
            pl.BlockSpec((f, h), lambda i: (0, 0)),
        ],
        out_specs=pl.BlockSpec((tile, h), lambda i: (i, 0)),
        compiler_params=pltpu.CompilerParams(
            dimension_semantics=("parallel",),
            vmem_limit_bytes=_VMEM_LIMIT,
        ),
    )(x, w_bf16)


def _z_body(x_ref, w_ref, adj_ref, z_ref, s_ref):
    t = pl.program_id(1)

    @pl.when(t == 0)
    def _():
        s_ref[...] = jnp.dot(
            x_ref[...].astype(jnp.bfloat16), w_ref[...],
            preferred_element_type=jnp.float32,
        ).astype(jnp.bfloat16)

    z_ref[...] = jnp.maximum(
        jnp.dot(
            adj_ref[...].astype(jnp.bfloat16), s_ref[...],
            preferred_element_type=jnp.float32,
        ),
        0.0,
    ).astype(jnp.bfloat16)


def _z_pallas(adj, x, w_bf16, *, tile_i):
    n = adj.shape[0]
    f = x.shape[1]
    h = w_bf16.shape[1]
    steps = n // tile_i
    return pl.pallas_call(
        _z_body,
        out_shape=jax.ShapeDtypeStruct((n, h), jnp.bfloat16),
        grid=(2, steps // 2),
        in_specs=[
            pl.BlockSpec((n, f), lambda c, t: (0, 0)),
            pl.BlockSpec((f, h), lambda c, t: (0, 0)),
            pl.BlockSpec((tile_i, n),
                         lambda c, t: (c * (pl.num_programs(1)) + t, 0)),
        ],
        out_specs=pl.BlockSpec(
            (tile_i, h), lambda c, t: (c * (pl.num_programs(1)) + t, 0)),
        scratch_shapes=[pltpu.VMEM((n, h), jnp.bfloat16)],
        compiler_params=pltpu.CompilerParams(
            dimension_semantics=("parallel", "arbitrary"),
            vmem_limit_bytes=_VMEM_LIMIT,
        ),
    )(x, w_bf16, adj)


def _dec_body(zr_ref, zc_ref, o_ref):
    logits = lax.dot_general(
        zr_ref[...], zc_ref[...],
        dimension_numbers=(((1,), (1,)), ((), ())),
        preferred_element_type=jnp.float32,
    )
    o_ref[...] = jax.nn.sigmoid(logits)


def _decoder(z, *, tile_i, tile_j):
    n, h = z.shape
    return pl.pallas_call(
        _dec_body,
        out_shape=jax.ShapeDtypeStruct((n, n), jnp.float32),
        grid=(n // tile_i, n // tile_j),
        in_specs=[
            pl.BlockSpec((tile_i, h), lambda i, j: (i, 0)),
            pl.BlockSpec((tile_j, h), lambda i, j: (j, 0)),
        ],
        out_specs=pl.BlockSpec((tile_i, tile_j), lambda i, j: (i, j)),
        compiler_params=pltpu.CompilerParams(
            dimension_semantics=("parallel", "parallel"),
            vmem_limit_bytes=_VMEM_LIMIT,
        ),
    )(z, z)


def kernel(atac_feature, scrna_feature, adj_norm, edge_attr, gc1_weight):
    del atac_feature, edge_attr

    n = adj_norm.shape[0]
    x = scrna_feature.astype(jnp.float32)
    adj = adj_norm.astype(jnp.float32)
    w_bf16 = gc1_weight.astype(jnp.bfloat16)

    pad = _round_up(n, 1024) - n
    if pad:
        adj = jnp.pad(adj, ((0, pad), (0, pad)))
        x = jnp.pad(x, ((0, pad), (0, 0)))
    n_p = n + pad

    z = _z_pallas(adj, x, w_bf16, tile_i=n_p // 8)
    a_pred = _decoder(z, tile_i=2048, tile_j=1024)
    return a_pred[:n, :n]
